# baseline (speedup 1.0000x reference)
import functools
import math

import jax
import jax.numpy as jnp
from jax import lax
from jax.experimental import pallas as pl
from jax.experimental.pallas import tpu as pltpu
from jax.experimental.pallas import tpu_sc as plsc

_NC = 80
_GRIDS = (80, 40, 20)
_NT = 3000
_NTP = 3072
_BATCH = 8
_NA = 3
_BAL = (4.0, 1.0, 0.4)
_HYP_BOX, _HYP_OBJ, _HYP_CLS = 0.05, 1.0, 0.5
_NCELLS = tuple(_BATCH * _NA * g * g for g in _GRIDS)
_ROWS_W = _NTP // 32
_IDX_R = _NTP // 128


_EX = ((4800, 200, 32), (1200, 200, 32), (600, 200, 16))
_ECHUNK = 200


def _sc_body(p0f, p1f, p2f, gidx0, gidx1, gidx2, fidx0, fidx1, fidx2, tval,
             g0, g1, g2, w0, w1, w2, c0, c1, c2, ob0, ob1, ob2,
             idx_v, rows_v, fidx_v, tval_v, cwin_v, wout_v, slab_v, obj_v,
             sem, sem2):
    cid = lax.axis_index("c")
    sid = lax.axis_index("s")
    wid = sid * 2 + cid
    base = wid * _ROWS_W
    levels = ((p0f, gidx0, fidx0, g0, w0, c0, ob0),
              (p1f, gidx1, fidx1, g1, w1, c1, ob1),
              (p2f, gidx2, fidx2, g2, w2, c2, ob2))
    for lvl in range(3):
        p, gidx, fidx, g_out, w_out, c_buf, ob_out = levels[lvl]

        nw_rows, chunk, nworkers = _EX[lvl]
        ngrp = (chunk + 15) // 16
        nch = nw_rows // chunk

        @pl.when(wid < nworkers)
        def _extract(p=p, ob_out=ob_out, nw_rows=nw_rows, chunk=chunk,
                     ngrp=ngrp, nch=nch):
            base_r = wid * nw_rows

            def extract_chunk(c, buf):
                def grp(k, _):
                    bufs = jnp.full((16,), buf, jnp.int32)
                    rows = jnp.minimum(
                        lax.iota(jnp.int32, 16) + k * 16, chunk - 1)
                    cols = jnp.full((16,), 4, jnp.int32)
                    vals = plsc.load_gather(slab_v, [bufs, rows, cols])
                    obj_v[pl.ds(c * chunk + k * 16, 16)] = vals
                    return 0

                lax.fori_loop(0, ngrp, grp, 0)

            def drain(s):
                pltpu.make_async_copy(p.at[pl.ds(base_r, chunk)],
                                      slab_v.at[0, pl.ds(0, chunk)], s).wait()

            pltpu.async_copy(p.at[pl.ds(base_r, chunk)], slab_v.at[0, pl.ds(0, chunk)], sem)

            def pair_body(h, _):
                c0 = 2 * h

                @pl.when(c0 + 1 < nch)
                def _():
                    pltpu.async_copy(
                        p.at[pl.ds(base_r + (c0 + 1) * chunk, chunk)],
                        slab_v.at[1, pl.ds(0, chunk)], sem2)

                drain(sem)
                extract_chunk(c0, 0)

                @pl.when(c0 + 2 < nch)
                def _():
                    pltpu.async_copy(
                        p.at[pl.ds(base_r + (c0 + 2) * chunk, chunk)],
                        slab_v.at[0, pl.ds(0, chunk)], sem)

                @pl.when(c0 + 1 < nch)
                def _():
                    drain(sem2)
                    extract_chunk(c0 + 1, 1)

                return 0

            lax.fori_loop(0, (nch + 1) // 2, pair_body, 0)
            pltpu.sync_copy(obj_v.at[pl.ds(0, nw_rows)],
                            ob_out.at[pl.ds(base_r, nw_rows)])
        pltpu.sync_copy(gidx.at[pl.ds(base, _ROWS_W)], idx_v)

        def row_dmas(j, _, p=p, rows_v=rows_v, idx_v=idx_v, sem=sem):
            v = idx_v[pl.ds(j * 16, 16)]
            for k in range(16):
                pltpu.async_copy(p.at[pl.ds(v[k], 1)],
                                 rows_v.at[pl.ds(j * 16 + k, 1)], sem)
            return 0

        lax.fori_loop(0, _ROWS_W // 16, row_dmas, 0)
        pltpu.make_async_copy(p.at[pl.ds(0, _ROWS_W)], rows_v, sem).wait()
        pltpu.sync_copy(rows_v, g_out.at[pl.ds(base, _ROWS_W)])

        @pl.when(wid == 29 + lvl)
        def _dedup(fidx=fidx, w_out=w_out, c_buf=c_buf):
            pltpu.sync_copy(fidx, fidx_v)
            pltpu.sync_copy(tval, tval_v)
            for j in range(_IDX_R):
                pltpu.async_copy(tval_v.at[j], c_buf.at[fidx_v.at[j]],
                                 sem).wait()
            for j in range(_IDX_R):
                pltpu.async_copy(c_buf.at[fidx_v.at[j]], cwin_v.at[j],
                                 sem).wait()
            for j in range(_IDX_R):
                for k in range(8):
                    sl = pl.ds(k * 16, 16)
                    eq = cwin_v[j, sl] == tval_v[j, sl]
                    wout_v[j, sl] = jnp.where(eq, 1, 0).astype(jnp.int32)
            pltpu.sync_copy(wout_v, w_out)


def _sc_gather(p0f, p1f, p2f, gidx, fidx, tval):
    mesh = plsc.VectorSubcoreMesh(core_axis_name="c", subcore_axis_name="s")
    out_type = (
        jax.ShapeDtypeStruct((_NTP, 85), jnp.float32),
        jax.ShapeDtypeStruct((_NTP, 85), jnp.float32),
        jax.ShapeDtypeStruct((_NTP, 85), jnp.float32),
        jax.ShapeDtypeStruct((_IDX_R, 128), jnp.int32),
        jax.ShapeDtypeStruct((_IDX_R, 128), jnp.int32),
        jax.ShapeDtypeStruct((_IDX_R, 128), jnp.int32),
        jax.ShapeDtypeStruct((_NCELLS[0] + 128,), jnp.int32),
        jax.ShapeDtypeStruct((_NCELLS[1] + 128,), jnp.int32),
        jax.ShapeDtypeStruct((_NCELLS[2] + 128,), jnp.int32),
        jax.ShapeDtypeStruct((_NCELLS[0],), jnp.float32),
        jax.ShapeDtypeStruct((_NCELLS[1],), jnp.float32),
        jax.ShapeDtypeStruct((_NCELLS[2],), jnp.float32),
    )
    scratch = (
        pltpu.VMEM((_ROWS_W,), jnp.int32),
        pltpu.VMEM((_ROWS_W, 85), jnp.float32),
        pltpu.VMEM((_IDX_R, 128), jnp.int32),
        pltpu.VMEM((_IDX_R, 128), jnp.int32),
        pltpu.VMEM((_IDX_R, 128), jnp.int32),
        pltpu.VMEM((_IDX_R, 128), jnp.int32),
        pltpu.VMEM((2, _ECHUNK, 85), jnp.float32),
        pltpu.VMEM((4816,), jnp.float32),
        pltpu.SemaphoreType.DMA,
        pltpu.SemaphoreType.DMA,
    )
    fn = pl.kernel(_sc_body, out_type=out_type, mesh=mesh,
                   scratch_types=scratch,
                   compiler_params=pltpu.CompilerParams(
                       needs_layout_passes=False))
    return fn(p0f, p1f, p2f, gidx[0], gidx[1], gidx[2],
              fidx[0], fidx[1], fidx[2], tval)


def _softplus(x):
    return jnp.maximum(x, 0.0) + jnp.log1p(jnp.exp(-jnp.abs(x)))


_OBJ2D = tuple((n // 128, 128) for n in _NCELLS)


def _atan_pos(z):
    inv = 1.0 / z
    r = jnp.minimum(z, inv)
    r2 = r * r
    p = -0.0117212
    p = p * r2 + 0.05265332
    p = p * r2 + -0.11643287
    p = p * r2 + 0.19354346
    p = p * r2 + -0.33262347
    p = p * r2 + 0.99997726
    a = p * r
    return jnp.where(z <= 1.0, a, (math.pi / 2) - a)


def _ciou(px, py, pw, ph, tx, ty, tw, th, eps=1e-7):
    b1x1, b1x2 = px - pw * 0.5, px + pw * 0.5
    b1y1, b1y2 = py - ph * 0.5, py + ph * 0.5
    b2x1, b2x2 = tx - tw * 0.5, tx + tw * 0.5
    b2y1, b2y2 = ty - th * 0.5, ty + th * 0.5
    iw = jnp.clip(jnp.minimum(b1x2, b2x2) - jnp.maximum(b1x1, b2x1), 0.0, None)
    ih = jnp.clip(jnp.minimum(b1y2, b2y2) - jnp.maximum(b1y1, b2y1), 0.0, None)
    inter = iw * ih
    union = pw * ph + tw * th - inter + eps
    iou = inter / union
    cw = jnp.maximum(b1x2, b2x2) - jnp.minimum(b1x1, b2x1)
    ch = jnp.maximum(b1y2, b2y2) - jnp.minimum(b1y1, b2y1)
    c2 = cw * cw + ch * ch + eps
    rho2 = ((b2x1 + b2x2 - b1x1 - b1x2) ** 2
            + (b2y1 + b2y2 - b1y1 - b1y2) ** 2) * 0.25
    v = (4.0 / math.pi**2) * (_atan_pos(tw / th) - _atan_pos(pw / ph)) ** 2
    alpha = v / (v - iou + (1.0 + eps))
    return iou - (rho2 / c2 + v * alpha)


def _fused_body(ob0, ob1, ob2, g0, g1, g2, w0, w1, w2, tb0, tb1, tb2,
                an0, an1, an2, tc0, tc1, tc2, out_ref):
    ob_refs = (ob0, ob1, ob2)
    g_refs = (g0, g1, g2)
    w_refs = (w0, w1, w2)
    tb_refs = (tb0, tb1, tb2)
    an_refs = (an0, an1, an2)
    tc_refs = (tc0, tc1, tc2)
    def pk(col):
        return jnp.reshape(col, (_NTP // 128, 128))

    valid = (lax.broadcasted_iota(jnp.int32, (_NTP, 1), 0)
             < _NT).astype(jnp.float32)
    validp = pk(valid)
    col = lax.broadcasted_iota(jnp.int32, (_NTP, _NC), 1)
    lbox = 0.0
    lobj = 0.0
    lcls = 0.0
    for lvl in range(3):
        dsum = jnp.sum(_softplus(ob_refs[lvl][:]))
        ps = g_refs[lvl][:]
        tb = tb_refs[lvl][:]
        an = an_refs[lvl][:]
        tcv = tc_refs[lvl][:]
        wfp = pk(w_refs[lvl][:])
        sx = jax.nn.sigmoid(pk(ps[:, 0:1])) * 2.0 - 0.5
        sy = jax.nn.sigmoid(pk(ps[:, 1:2])) * 2.0 - 0.5
        sw = jax.nn.sigmoid(pk(ps[:, 2:3])) * 2.0
        sh = jax.nn.sigmoid(pk(ps[:, 3:4])) * 2.0
        pw = sw * sw * pk(an[:, 0:1])
        ph = sh * sh * pk(an[:, 1:2])
        iou = _ciou(sx, sy, pw, ph,
                    pk(tb[:, 0:1]), pk(tb[:, 1:2]),
                    pk(tb[:, 2:3]), pk(tb[:, 3:4]))
        lbox = lbox + jnp.sum((1.0 - iou) * validp) / _NT
        objp = pk(ps[:, 4:5])
        corr = jnp.sum(objp * jnp.clip(iou, 0.0, None) * wfp * validp)
        lobj = lobj + (dsum - corr) / _NCELLS[lvl] * _BAL[lvl]
        cls = ps[:, 5:85]
        keep = ((tcv > 0).astype(jnp.float32)) * valid
        row_sp = jnp.sum(_softplus(cls), axis=1, keepdims=True)
        sel = jnp.sum(jnp.where(col == (tcv - 1), cls, 0.0), axis=1,
                      keepdims=True)
        nk = jnp.sum(keep)
        csum = jnp.sum((row_sp - sel) * keep)
        lcls = lcls + jnp.where(nk > 0.0,
                                csum / (jnp.maximum(nk, 1.0) * _NC), 0.0)
    out_ref[0, 0] = (lbox * _HYP_BOX + lobj * _HYP_OBJ
                     + lcls * _HYP_CLS) * _BATCH


def _fused(obs, gs, ws, tbs, ans, tcs):
    def const2(shape):
        return pl.BlockSpec(shape, lambda: (0, 0))

    in_specs = (
        [const2(_OBJ2D[0]), const2(_OBJ2D[1]), const2(_OBJ2D[2])]
        + [const2((_NTP, 85))] * 3 + [const2((_NTP, 1))] * 3
        + [const2((_NTP, 4))] * 3 + [const2((_NTP, 2))] * 3
        + [const2((_NTP, 1))] * 3
    )
    return pl.pallas_call(
        _fused_body,
        in_specs=in_specs,
        out_specs=pl.BlockSpec((1, 1), lambda: (0, 0),
                               memory_space=pltpu.SMEM),
        out_shape=jax.ShapeDtypeStruct((1, 1), jnp.float32),
    )(*obs, *gs, *ws, *tbs, *ans, *tcs)


def kernel(p0, p1, p2, tbox0, tbox1, tbox2, anch0, anch1, anch2,
           tcls0, tcls1, tcls2, b0, b1, b2, a0, a1, a2,
           gj0, gj1, gj2, gi0, gi1, gi2):
    pf = [p0.reshape(-1, 85), p1.reshape(-1, 85), p2.reshape(-1, 85)]
    pad = _NTP - _NT

    gidx, fidx = [], []
    for lvl, (b, a, gj, gi) in enumerate(((b0, a0, gj0, gi0),
                                          (b1, a1, gj1, gi1),
                                          (b2, a2, gj2, gi2))):
        g = _GRIDS[lvl]
        f = ((b.astype(jnp.int32) * _NA + a.astype(jnp.int32)) * g
             + gj.astype(jnp.int32)) * g + gi.astype(jnp.int32)
        f = jnp.concatenate(
            [f, jnp.full((pad,), _NCELLS[lvl], jnp.int32)])
        gidx.append(jnp.minimum(f, _NCELLS[lvl] - 1))
        fidx.append(f.reshape(_IDX_R, 128))
    tval = jnp.arange(_NTP, dtype=jnp.int32).reshape(_IDX_R, 128)

    sc_out = _sc_gather(pf[0], pf[1], pf[2], gidx, fidx, tval)
    gs = sc_out[0:3]
    ws = [w.reshape(_NTP, 1).astype(jnp.float32) for w in sc_out[3:6]]
    obs = [o.reshape(s) for o, s in zip(sc_out[9:12], _OBJ2D)]

    def padf(x, v):
        return jnp.concatenate(
            [x.astype(jnp.float32),
             jnp.full((pad, x.shape[1]), v, jnp.float32)])

    tbs = [padf(t, 1.0) for t in (tbox0, tbox1, tbox2)]
    ans = [padf(t, 1.0) for t in (anch0, anch1, anch2)]
    tcs = [jnp.concatenate([t.astype(jnp.int32),
                            jnp.zeros((pad,), jnp.int32)]).reshape(_NTP, 1)
           for t in (tcls0, tcls1, tcls2)]

    total = _fused(obs, gs, ws, tbs, ans, tcs)
    return total.reshape(())


# speedup vs baseline: 1.0995x; 1.0995x over previous
import functools
import math

import jax
import jax.numpy as jnp
from jax import lax
from jax.experimental import pallas as pl
from jax.experimental.pallas import tpu as pltpu
from jax.experimental.pallas import tpu_sc as plsc

_NC = 80
_GRIDS = (80, 40, 20)
_NT = 3000
_NTP = 3072
_BATCH = 8
_NA = 3
_BAL = (4.0, 1.0, 0.4)
_HYP_BOX, _HYP_OBJ, _HYP_CLS = 0.05, 1.0, 0.5
_NCELLS = tuple(_BATCH * _NA * g * g for g in _GRIDS)
_ROWS_W = _NTP // 32
_IDX_R = _NTP // 128


_EX = ((2400, 200, 32), (1200, 200, 32), (600, 200, 16))
_ECHUNK = 200
_SC0 = _EX[0][0] * _EX[0][2]
_TCBLK = 1920
_TCNB = (_NCELLS[0] - _SC0) // _TCBLK


def _sc_body(p0f, p1f, p2f, gidx0, gidx1, gidx2, fidx0, fidx1, fidx2, tval,
             g0, g1, g2, w0, w1, w2, c0, c1, c2, ob0, ob1, ob2,
             idx_v, rows_v, fidx_v, tval_v, cwin_v, wout_v, slab_v, obj_v,
             sem, sem2):
    cid = lax.axis_index("c")
    sid = lax.axis_index("s")
    wid = sid * 2 + cid
    base = wid * _ROWS_W
    levels = ((p0f, gidx0, fidx0, g0, w0, c0, ob0),
              (p1f, gidx1, fidx1, g1, w1, c1, ob1),
              (p2f, gidx2, fidx2, g2, w2, c2, ob2))
    for lvl in range(3):
        p, gidx, fidx, g_out, w_out, c_buf, ob_out = levels[lvl]

        nw_rows, chunk, nworkers = _EX[lvl]
        ngrp = (chunk + 15) // 16
        nch = nw_rows // chunk

        @pl.when(wid < nworkers)
        def _extract(p=p, ob_out=ob_out, nw_rows=nw_rows, chunk=chunk,
                     ngrp=ngrp, nch=nch):
            base_r = wid * nw_rows

            def extract_chunk(c, buf):
                def grp(k, _):
                    bufs = jnp.full((16,), buf, jnp.int32)
                    rows = jnp.minimum(
                        lax.iota(jnp.int32, 16) + k * 16, chunk - 1)
                    cols = jnp.full((16,), 4, jnp.int32)
                    vals = plsc.load_gather(slab_v, [bufs, rows, cols])
                    obj_v[pl.ds(c * chunk + k * 16, 16)] = vals
                    return 0

                lax.fori_loop(0, ngrp, grp, 0)

            def drain(s):
                pltpu.make_async_copy(p.at[pl.ds(base_r, chunk)],
                                      slab_v.at[0, pl.ds(0, chunk)], s).wait()

            pltpu.async_copy(p.at[pl.ds(base_r, chunk)], slab_v.at[0, pl.ds(0, chunk)], sem)

            def pair_body(h, _):
                c0 = 2 * h

                @pl.when(c0 + 1 < nch)
                def _():
                    pltpu.async_copy(
                        p.at[pl.ds(base_r + (c0 + 1) * chunk, chunk)],
                        slab_v.at[1, pl.ds(0, chunk)], sem2)

                drain(sem)
                extract_chunk(c0, 0)

                @pl.when(c0 + 2 < nch)
                def _():
                    pltpu.async_copy(
                        p.at[pl.ds(base_r + (c0 + 2) * chunk, chunk)],
                        slab_v.at[0, pl.ds(0, chunk)], sem)

                @pl.when(c0 + 1 < nch)
                def _():
                    drain(sem2)
                    extract_chunk(c0 + 1, 1)

                return 0

            lax.fori_loop(0, (nch + 1) // 2, pair_body, 0)
            pltpu.sync_copy(obj_v.at[pl.ds(0, nw_rows)],
                            ob_out.at[pl.ds(base_r, nw_rows)])
        pltpu.sync_copy(gidx.at[pl.ds(base, _ROWS_W)], idx_v)

        def row_dmas(j, _, p=p, rows_v=rows_v, idx_v=idx_v, sem=sem):
            v = idx_v[pl.ds(j * 16, 16)]
            for k in range(16):
                pltpu.async_copy(p.at[pl.ds(v[k], 1)],
                                 rows_v.at[pl.ds(j * 16 + k, 1)], sem)
            return 0

        lax.fori_loop(0, _ROWS_W // 16, row_dmas, 0)
        pltpu.make_async_copy(p.at[pl.ds(0, _ROWS_W)], rows_v, sem).wait()
        pltpu.sync_copy(rows_v, g_out.at[pl.ds(base, _ROWS_W)])

        @pl.when(wid == 29 + lvl)
        def _dedup(fidx=fidx, w_out=w_out, c_buf=c_buf):
            pltpu.sync_copy(fidx, fidx_v)
            pltpu.sync_copy(tval, tval_v)
            for j in range(_IDX_R):
                pltpu.async_copy(tval_v.at[j], c_buf.at[fidx_v.at[j]], sem)
            pltpu.make_async_copy(fidx, tval_v, sem).wait()
            for j in range(_IDX_R):
                pltpu.async_copy(c_buf.at[fidx_v.at[j]], cwin_v.at[j], sem)
            pltpu.make_async_copy(fidx, cwin_v, sem).wait()
            for j in range(_IDX_R):
                for k in range(8):
                    sl = pl.ds(k * 16, 16)
                    eq = cwin_v[j, sl] == tval_v[j, sl]
                    wout_v[j, sl] = jnp.where(eq, 1, 0).astype(jnp.int32)
            pltpu.sync_copy(wout_v, w_out)


def _sc_gather(p0f, p1f, p2f, gidx, fidx, tval):
    mesh = plsc.VectorSubcoreMesh(core_axis_name="c", subcore_axis_name="s")
    out_type = (
        jax.ShapeDtypeStruct((_NTP, 85), jnp.float32),
        jax.ShapeDtypeStruct((_NTP, 85), jnp.float32),
        jax.ShapeDtypeStruct((_NTP, 85), jnp.float32),
        jax.ShapeDtypeStruct((_IDX_R, 128), jnp.int32),
        jax.ShapeDtypeStruct((_IDX_R, 128), jnp.int32),
        jax.ShapeDtypeStruct((_IDX_R, 128), jnp.int32),
        jax.ShapeDtypeStruct((_NCELLS[0] + 128,), jnp.int32),
        jax.ShapeDtypeStruct((_NCELLS[1] + 128,), jnp.int32),
        jax.ShapeDtypeStruct((_NCELLS[2] + 128,), jnp.int32),
        jax.ShapeDtypeStruct((_SC0,), jnp.float32),
        jax.ShapeDtypeStruct((_NCELLS[1],), jnp.float32),
        jax.ShapeDtypeStruct((_NCELLS[2],), jnp.float32),
    )
    scratch = (
        pltpu.VMEM((_ROWS_W,), jnp.int32),
        pltpu.VMEM((_ROWS_W, 85), jnp.float32),
        pltpu.VMEM((_IDX_R, 128), jnp.int32),
        pltpu.VMEM((_IDX_R, 128), jnp.int32),
        pltpu.VMEM((_IDX_R, 128), jnp.int32),
        pltpu.VMEM((_IDX_R, 128), jnp.int32),
        pltpu.VMEM((2, _ECHUNK, 85), jnp.float32),
        pltpu.VMEM((4816,), jnp.float32),
        pltpu.SemaphoreType.DMA,
        pltpu.SemaphoreType.DMA,
    )
    fn = pl.kernel(_sc_body, out_type=out_type, mesh=mesh,
                   scratch_types=scratch,
                   compiler_params=pltpu.CompilerParams(
                       needs_layout_passes=False))
    return fn(p0f, p1f, p2f, gidx[0], gidx[1], gidx[2],
              fidx[0], fidx[1], fidx[2], tval)


def _softplus(x):
    return jnp.maximum(x, 0.0) + jnp.log1p(jnp.exp(-jnp.abs(x)))


_OBJ2D = ((_SC0 // 128, 128), (_NCELLS[1] // 128, 128),
          (_NCELLS[2] // 128, 128))


def _obj_half_body(p_ref, o_ref):
    x = p_ref[:, 4:5]
    s = jnp.sum(_softplus(x))

    @pl.when(pl.program_id(0) == 0)
    def _():
        o_ref[0, 0] = 0.0

    o_ref[0, 0] += s


def _obj_half(pf0):
    return pl.pallas_call(
        _obj_half_body,
        grid=(_TCNB,),
        in_specs=[pl.BlockSpec((_TCBLK, 85),
                               lambda i: (_SC0 // _TCBLK + i, 0))],
        out_specs=pl.BlockSpec((1, 1), lambda i: (0, 0),
                               memory_space=pltpu.SMEM),
        out_shape=jax.ShapeDtypeStruct((1, 1), jnp.float32),
    )(pf0)


def _atan_pos(z):
    inv = 1.0 / z
    r = jnp.minimum(z, inv)
    r2 = r * r
    p = -0.0117212
    p = p * r2 + 0.05265332
    p = p * r2 + -0.11643287
    p = p * r2 + 0.19354346
    p = p * r2 + -0.33262347
    p = p * r2 + 0.99997726
    a = p * r
    return jnp.where(z <= 1.0, a, (math.pi / 2) - a)


def _ciou(px, py, pw, ph, tx, ty, tw, th, eps=1e-7):
    b1x1, b1x2 = px - pw * 0.5, px + pw * 0.5
    b1y1, b1y2 = py - ph * 0.5, py + ph * 0.5
    b2x1, b2x2 = tx - tw * 0.5, tx + tw * 0.5
    b2y1, b2y2 = ty - th * 0.5, ty + th * 0.5
    iw = jnp.clip(jnp.minimum(b1x2, b2x2) - jnp.maximum(b1x1, b2x1), 0.0, None)
    ih = jnp.clip(jnp.minimum(b1y2, b2y2) - jnp.maximum(b1y1, b2y1), 0.0, None)
    inter = iw * ih
    union = pw * ph + tw * th - inter + eps
    iou = inter / union
    cw = jnp.maximum(b1x2, b2x2) - jnp.minimum(b1x1, b2x1)
    ch = jnp.maximum(b1y2, b2y2) - jnp.minimum(b1y1, b2y1)
    c2 = cw * cw + ch * ch + eps
    rho2 = ((b2x1 + b2x2 - b1x1 - b1x2) ** 2
            + (b2y1 + b2y2 - b1y1 - b1y2) ** 2) * 0.25
    v = (4.0 / math.pi**2) * (_atan_pos(tw / th) - _atan_pos(pw / ph)) ** 2
    alpha = v / (v - iou + (1.0 + eps))
    return iou - (rho2 / c2 + v * alpha)


def _fused_body(ob0, ob1, ob2, d0h, g0, g1, g2, w0, w1, w2, tb0, tb1, tb2,
                an0, an1, an2, tc0, tc1, tc2, out_ref):
    ob_refs = (ob0, ob1, ob2)
    g_refs = (g0, g1, g2)
    w_refs = (w0, w1, w2)
    tb_refs = (tb0, tb1, tb2)
    an_refs = (an0, an1, an2)
    tc_refs = (tc0, tc1, tc2)
    def pk(col):
        return jnp.reshape(col, (_NTP // 128, 128))

    valid = (lax.broadcasted_iota(jnp.int32, (_NTP, 1), 0)
             < _NT).astype(jnp.float32)
    validp = pk(valid)
    col = lax.broadcasted_iota(jnp.int32, (_NTP, _NC), 1)
    lbox = 0.0
    lobj = 0.0
    lcls = 0.0
    for lvl in range(3):
        dsum = jnp.sum(_softplus(ob_refs[lvl][:]))
        if lvl == 0:
            dsum = dsum + d0h[0, 0]
        ps = g_refs[lvl][:]
        tb = tb_refs[lvl][:]
        an = an_refs[lvl][:]
        tcv = tc_refs[lvl][:]
        wfp = pk(w_refs[lvl][:])
        sx = jax.nn.sigmoid(pk(ps[:, 0:1])) * 2.0 - 0.5
        sy = jax.nn.sigmoid(pk(ps[:, 1:2])) * 2.0 - 0.5
        sw = jax.nn.sigmoid(pk(ps[:, 2:3])) * 2.0
        sh = jax.nn.sigmoid(pk(ps[:, 3:4])) * 2.0
        pw = sw * sw * pk(an[:, 0:1])
        ph = sh * sh * pk(an[:, 1:2])
        iou = _ciou(sx, sy, pw, ph,
                    pk(tb[:, 0:1]), pk(tb[:, 1:2]),
                    pk(tb[:, 2:3]), pk(tb[:, 3:4]))
        lbox = lbox + jnp.sum((1.0 - iou) * validp) / _NT
        objp = pk(ps[:, 4:5])
        corr = jnp.sum(objp * jnp.clip(iou, 0.0, None) * wfp * validp)
        lobj = lobj + (dsum - corr) / _NCELLS[lvl] * _BAL[lvl]
        cls = ps[:, 5:85]
        keep = ((tcv > 0).astype(jnp.float32)) * valid
        row_sp = jnp.sum(_softplus(cls), axis=1, keepdims=True)
        sel = jnp.sum(jnp.where(col == (tcv - 1), cls, 0.0), axis=1,
                      keepdims=True)
        nk = jnp.sum(keep)
        csum = jnp.sum((row_sp - sel) * keep)
        lcls = lcls + jnp.where(nk > 0.0,
                                csum / (jnp.maximum(nk, 1.0) * _NC), 0.0)
    out_ref[0, 0] = (lbox * _HYP_BOX + lobj * _HYP_OBJ
                     + lcls * _HYP_CLS) * _BATCH


def _fused(obs, d0h, gs, ws, tbs, ans, tcs):
    def const2(shape):
        return pl.BlockSpec(shape, lambda: (0, 0))

    sspec = pl.BlockSpec((1, 1), lambda: (0, 0), memory_space=pltpu.SMEM)
    in_specs = (
        [const2(_OBJ2D[0]), const2(_OBJ2D[1]), const2(_OBJ2D[2]), sspec]
        + [const2((_NTP, 85))] * 3 + [const2((_NTP, 1))] * 3
        + [const2((_NTP, 4))] * 3 + [const2((_NTP, 2))] * 3
        + [const2((_NTP, 1))] * 3
    )
    return pl.pallas_call(
        _fused_body,
        in_specs=in_specs,
        out_specs=sspec,
        out_shape=jax.ShapeDtypeStruct((1, 1), jnp.float32),
    )(*obs, d0h, *gs, *ws, *tbs, *ans, *tcs)


def kernel(p0, p1, p2, tbox0, tbox1, tbox2, anch0, anch1, anch2,
           tcls0, tcls1, tcls2, b0, b1, b2, a0, a1, a2,
           gj0, gj1, gj2, gi0, gi1, gi2):
    pf = [p0.reshape(-1, 85), p1.reshape(-1, 85), p2.reshape(-1, 85)]
    pad = _NTP - _NT

    gidx, fidx = [], []
    for lvl, (b, a, gj, gi) in enumerate(((b0, a0, gj0, gi0),
                                          (b1, a1, gj1, gi1),
                                          (b2, a2, gj2, gi2))):
        g = _GRIDS[lvl]
        f = ((b.astype(jnp.int32) * _NA + a.astype(jnp.int32)) * g
             + gj.astype(jnp.int32)) * g + gi.astype(jnp.int32)
        f = jnp.concatenate(
            [f, jnp.full((pad,), _NCELLS[lvl], jnp.int32)])
        gidx.append(jnp.minimum(f, _NCELLS[lvl] - 1))
        fidx.append(f.reshape(_IDX_R, 128))
    tval = jnp.arange(_NTP, dtype=jnp.int32).reshape(_IDX_R, 128)

    sc_out = _sc_gather(pf[0], pf[1], pf[2], gidx, fidx, tval)
    gs = sc_out[0:3]
    ws = [w.reshape(_NTP, 1).astype(jnp.float32) for w in sc_out[3:6]]
    obs = [o.reshape(s) for o, s in zip(sc_out[9:12], _OBJ2D)]

    def padf(x, v):
        return jnp.concatenate(
            [x.astype(jnp.float32),
             jnp.full((pad, x.shape[1]), v, jnp.float32)])

    tbs = [padf(t, 1.0) for t in (tbox0, tbox1, tbox2)]
    ans = [padf(t, 1.0) for t in (anch0, anch1, anch2)]
    tcs = [jnp.concatenate([t.astype(jnp.int32),
                            jnp.zeros((pad,), jnp.int32)]).reshape(_NTP, 1)
           for t in (tcls0, tcls1, tcls2)]

    total = _fused(obs, _obj_half(pf[0]), gs, ws, tbs, ans, tcs)
    return total.reshape(())


# speedup vs baseline: 1.1835x; 1.0764x over previous
import functools
import math

import jax
import jax.numpy as jnp
from jax import lax
from jax.experimental import pallas as pl
from jax.experimental.pallas import tpu as pltpu
from jax.experimental.pallas import tpu_sc as plsc

_NC = 80
_GRIDS = (80, 40, 20)
_NT = 3000
_NTP = 3072
_BATCH = 8
_NA = 3
_BAL = (4.0, 1.0, 0.4)
_HYP_BOX, _HYP_OBJ, _HYP_CLS = 0.05, 1.0, 0.5
_NCELLS = tuple(_BATCH * _NA * g * g for g in _GRIDS)
_ROWS_W = _NTP // 32
_IDX_R = _NTP // 128


_EX = ((2400, 200, 32), (1200, 200, 32), (600, 200, 16))
_ECHUNK = 200
_SC0 = _EX[0][0] * _EX[0][2]
_TCBLK = 1920
_TCNB = (_NCELLS[0] - _SC0) // _TCBLK


def _sc_body(p0f, p1f, p2f, gidx0, gidx1, gidx2, fidx0, fidx1, fidx2, tval,
             g0, g1, g2, w0, w1, w2, c0, c1, c2, ob0, ob1, ob2,
             idx_v, rows_v, fidx_v, tval_v, cwin_v, wout_v, slab_v, obj_v,
             sem, sem2):
    cid = lax.axis_index("c")
    sid = lax.axis_index("s")
    wid = sid * 2 + cid
    base = wid * _ROWS_W
    levels = ((p0f, gidx0, fidx0, g0, w0, c0, ob0),
              (p1f, gidx1, fidx1, g1, w1, c1, ob1),
              (p2f, gidx2, fidx2, g2, w2, c2, ob2))
    for lvl in range(3):
        p, gidx, fidx, g_out, w_out, c_buf, ob_out = levels[lvl]

        nw_rows, chunk, nworkers = _EX[lvl]
        ngrp = (chunk + 15) // 16
        nch = nw_rows // chunk

        @pl.when(wid < nworkers)
        def _extract(p=p, ob_out=ob_out, nw_rows=nw_rows, chunk=chunk,
                     ngrp=ngrp, nch=nch):
            base_r = wid * nw_rows

            def extract_chunk(c, buf):
                def grp(k, _):
                    bufs = jnp.full((16,), buf, jnp.int32)
                    rows = jnp.minimum(
                        lax.iota(jnp.int32, 16) + k * 16, chunk - 1)
                    cols = jnp.full((16,), 4, jnp.int32)
                    vals = plsc.load_gather(slab_v, [bufs, rows, cols])
                    obj_v[pl.ds(c * chunk + k * 16, 16)] = vals
                    return 0

                lax.fori_loop(0, ngrp, grp, 0)

            def drain(s):
                pltpu.make_async_copy(p.at[pl.ds(base_r, chunk)],
                                      slab_v.at[0, pl.ds(0, chunk)], s).wait()

            pltpu.async_copy(p.at[pl.ds(base_r, chunk)], slab_v.at[0, pl.ds(0, chunk)], sem)

            def pair_body(h, _):
                c0 = 2 * h

                @pl.when(c0 + 1 < nch)
                def _():
                    pltpu.async_copy(
                        p.at[pl.ds(base_r + (c0 + 1) * chunk, chunk)],
                        slab_v.at[1, pl.ds(0, chunk)], sem2)

                drain(sem)
                extract_chunk(c0, 0)

                @pl.when(c0 + 2 < nch)
                def _():
                    pltpu.async_copy(
                        p.at[pl.ds(base_r + (c0 + 2) * chunk, chunk)],
                        slab_v.at[0, pl.ds(0, chunk)], sem)

                @pl.when(c0 + 1 < nch)
                def _():
                    drain(sem2)
                    extract_chunk(c0 + 1, 1)

                return 0

            lax.fori_loop(0, (nch + 1) // 2, pair_body, 0)
            pltpu.sync_copy(obj_v.at[pl.ds(0, nw_rows)],
                            ob_out.at[pl.ds(base_r, nw_rows)])
        pltpu.sync_copy(gidx.at[pl.ds(base, _ROWS_W)], idx_v)

        def row_dmas(j, _, p=p, rows_v=rows_v, idx_v=idx_v, sem=sem):
            v = idx_v[pl.ds(j * 16, 16)]
            for k in range(16):
                pltpu.async_copy(p.at[pl.ds(v[k], 1)],
                                 rows_v.at[pl.ds(j * 16 + k, 1)], sem)
            return 0

        lax.fori_loop(0, _ROWS_W // 16, row_dmas, 0)
        pltpu.make_async_copy(p.at[pl.ds(0, _ROWS_W)], rows_v, sem).wait()
        pltpu.sync_copy(rows_v, g_out.at[pl.ds(base, _ROWS_W)])

        @pl.when(wid == 29 + lvl)
        def _dedup(fidx=fidx, w_out=w_out, c_buf=c_buf):
            pltpu.sync_copy(fidx, fidx_v)
            pltpu.sync_copy(tval, tval_v)
            for j in range(_IDX_R):
                pltpu.async_copy(tval_v.at[j], c_buf.at[fidx_v.at[j]], sem)
            pltpu.make_async_copy(fidx, tval_v, sem).wait()
            for j in range(_IDX_R):
                pltpu.async_copy(c_buf.at[fidx_v.at[j]], cwin_v.at[j], sem)
            pltpu.make_async_copy(fidx, cwin_v, sem).wait()
            for j in range(_IDX_R):
                for k in range(8):
                    sl = pl.ds(k * 16, 16)
                    eq = cwin_v[j, sl] == tval_v[j, sl]
                    wout_v[j, sl] = jnp.where(eq, 1, 0).astype(jnp.int32)
            pltpu.sync_copy(wout_v, w_out)


def _sc_gather(p0f, p1f, p2f, gidx, fidx, tval):
    mesh = plsc.VectorSubcoreMesh(core_axis_name="c", subcore_axis_name="s")
    out_type = (
        jax.ShapeDtypeStruct((_NTP, 85), jnp.float32),
        jax.ShapeDtypeStruct((_NTP, 85), jnp.float32),
        jax.ShapeDtypeStruct((_NTP, 85), jnp.float32),
        jax.ShapeDtypeStruct((_IDX_R, 128), jnp.int32),
        jax.ShapeDtypeStruct((_IDX_R, 128), jnp.int32),
        jax.ShapeDtypeStruct((_IDX_R, 128), jnp.int32),
        jax.ShapeDtypeStruct((_NCELLS[0] + 128,), jnp.int32),
        jax.ShapeDtypeStruct((_NCELLS[1] + 128,), jnp.int32),
        jax.ShapeDtypeStruct((_NCELLS[2] + 128,), jnp.int32),
        jax.ShapeDtypeStruct((_SC0,), jnp.float32),
        jax.ShapeDtypeStruct((_NCELLS[1],), jnp.float32),
        jax.ShapeDtypeStruct((_NCELLS[2],), jnp.float32),
    )
    scratch = (
        pltpu.VMEM((_ROWS_W,), jnp.int32),
        pltpu.VMEM((_ROWS_W, 85), jnp.float32),
        pltpu.VMEM((_IDX_R, 128), jnp.int32),
        pltpu.VMEM((_IDX_R, 128), jnp.int32),
        pltpu.VMEM((_IDX_R, 128), jnp.int32),
        pltpu.VMEM((_IDX_R, 128), jnp.int32),
        pltpu.VMEM((2, _ECHUNK, 85), jnp.float32),
        pltpu.VMEM((4816,), jnp.float32),
        pltpu.SemaphoreType.DMA,
        pltpu.SemaphoreType.DMA,
    )
    fn = pl.kernel(_sc_body, out_type=out_type, mesh=mesh,
                   scratch_types=scratch,
                   compiler_params=pltpu.CompilerParams(
                       needs_layout_passes=False))
    return fn(p0f, p1f, p2f, gidx[0], gidx[1], gidx[2],
              fidx[0], fidx[1], fidx[2], tval)


def _softplus(x):
    return jnp.maximum(x, 0.0) + jnp.log1p(jnp.exp(-jnp.abs(x)))


_OBJ2D = ((_SC0 // 128, 128), (_NCELLS[1] // 128, 128),
          (_NCELLS[2] // 128, 128))


def _obj_half_body(p_ref, o_ref):
    x = p_ref[:, 4:5]
    s = jnp.sum(_softplus(x))

    @pl.when(pl.program_id(0) == 0)
    def _():
        o_ref[0, 0] = 0.0

    o_ref[0, 0] += s


def _obj_half(pf0):
    return pl.pallas_call(
        _obj_half_body,
        grid=(_TCNB,),
        in_specs=[pl.BlockSpec((_TCBLK, 85),
                               lambda i: (_SC0 // _TCBLK + i, 0))],
        out_specs=pl.BlockSpec((1, 1), lambda i: (0, 0),
                               memory_space=pltpu.SMEM),
        out_shape=jax.ShapeDtypeStruct((1, 1), jnp.float32),
    )(pf0)


def _atan_pos(z):
    inv = 1.0 / z
    r = jnp.minimum(z, inv)
    r2 = r * r
    p = -0.0117212
    p = p * r2 + 0.05265332
    p = p * r2 + -0.11643287
    p = p * r2 + 0.19354346
    p = p * r2 + -0.33262347
    p = p * r2 + 0.99997726
    a = p * r
    return jnp.where(z <= 1.0, a, (math.pi / 2) - a)


def _ciou(px, py, pw, ph, tx, ty, tw, th, eps=1e-7):
    b1x1, b1x2 = px - pw * 0.5, px + pw * 0.5
    b1y1, b1y2 = py - ph * 0.5, py + ph * 0.5
    b2x1, b2x2 = tx - tw * 0.5, tx + tw * 0.5
    b2y1, b2y2 = ty - th * 0.5, ty + th * 0.5
    iw = jnp.clip(jnp.minimum(b1x2, b2x2) - jnp.maximum(b1x1, b2x1), 0.0, None)
    ih = jnp.clip(jnp.minimum(b1y2, b2y2) - jnp.maximum(b1y1, b2y1), 0.0, None)
    inter = iw * ih
    union = pw * ph + tw * th - inter + eps
    iou = inter / union
    cw = jnp.maximum(b1x2, b2x2) - jnp.minimum(b1x1, b2x1)
    ch = jnp.maximum(b1y2, b2y2) - jnp.minimum(b1y1, b2y1)
    c2 = cw * cw + ch * ch + eps
    rho2 = ((b2x1 + b2x2 - b1x1 - b1x2) ** 2
            + (b2y1 + b2y2 - b1y1 - b1y2) ** 2) * 0.25
    v = (4.0 / math.pi**2) * (_atan_pos(tw / th) - _atan_pos(pw / ph)) ** 2
    alpha = v / (v - iou + (1.0 + eps))
    return iou - (rho2 / c2 + v * alpha)


def _fused_body(ob0, ob1, ob2, d0h, g0, g1, g2, w0, w1, w2, tb0, tb1, tb2,
                an0, an1, an2, tc0, tc1, tc2, out_ref):
    ob_refs = (ob0, ob1, ob2)
    g_refs = (g0, g1, g2)
    w_refs = (w0, w1, w2)
    tb_refs = (tb0, tb1, tb2)
    an_refs = (an0, an1, an2)
    tc_refs = (tc0, tc1, tc2)
    def pk(col):
        return jnp.reshape(col, (_NTP // 128, 128))

    valid = (lax.broadcasted_iota(jnp.int32, (_NTP, 1), 0)
             < _NT).astype(jnp.float32)
    validp = pk(valid)
    col = lax.broadcasted_iota(jnp.int32, (_NTP, _NC), 1)
    lbox = 0.0
    lobj = 0.0
    lcls = 0.0
    for lvl in range(3):
        dsum = jnp.sum(_softplus(ob_refs[lvl][:]))
        if lvl == 0:
            dsum = dsum + d0h[0, 0]
        ps = g_refs[lvl][:]
        tb = tb_refs[lvl][:]
        an = an_refs[lvl][:]
        tcv = tc_refs[lvl][:]
        wfp = pk(w_refs[lvl][:])
        sx = jax.nn.sigmoid(pk(ps[:, 0:1])) * 2.0 - 0.5
        sy = jax.nn.sigmoid(pk(ps[:, 1:2])) * 2.0 - 0.5
        sw = jax.nn.sigmoid(pk(ps[:, 2:3])) * 2.0
        sh = jax.nn.sigmoid(pk(ps[:, 3:4])) * 2.0
        pw = sw * sw * pk(an[:, 0:1])
        ph = sh * sh * pk(an[:, 1:2])
        iou = _ciou(sx, sy, pw, ph,
                    pk(tb[:, 0:1]), pk(tb[:, 1:2]),
                    pk(tb[:, 2:3]), pk(tb[:, 3:4]))
        lbox = lbox + jnp.sum((1.0 - iou) * validp) / _NT
        objp = pk(ps[:, 4:5])
        corr = jnp.sum(objp * jnp.clip(iou, 0.0, None) * wfp * validp)
        lobj = lobj + (dsum - corr) / _NCELLS[lvl] * _BAL[lvl]
        cls = ps[:, 5:85]
        keep = ((tcv > 0).astype(jnp.float32)) * valid
        row_sp = jnp.sum(_softplus(cls), axis=1, keepdims=True)
        sel = jnp.sum(jnp.where(col == (tcv - 1), cls, 0.0), axis=1,
                      keepdims=True)
        nk = jnp.sum(keep)
        csum = jnp.sum((row_sp - sel) * keep)
        lcls = lcls + jnp.where(nk > 0.0,
                                csum / (jnp.maximum(nk, 1.0) * _NC), 0.0)
    out_ref[0, 0] = (lbox * _HYP_BOX + lobj * _HYP_OBJ
                     + lcls * _HYP_CLS) * _BATCH


def _fused(obs, d0h, gs, ws, tbs, ans, tcs):
    def const2(shape):
        return pl.BlockSpec(shape, lambda: (0, 0))

    sspec = pl.BlockSpec((1, 1), lambda: (0, 0), memory_space=pltpu.SMEM)
    in_specs = (
        [const2(_OBJ2D[0]), const2(_OBJ2D[1]), const2(_OBJ2D[2]), sspec]
        + [const2((_NTP, 85))] * 3 + [const2((_NTP, 1))] * 3
        + [const2((_NTP, 4))] * 3 + [const2((_NTP, 2))] * 3
        + [const2((_NTP, 1))] * 3
    )
    return pl.pallas_call(
        _fused_body,
        in_specs=in_specs,
        out_specs=sspec,
        out_shape=jax.ShapeDtypeStruct((1, 1), jnp.float32),
    )(*obs, d0h, *gs, *ws, *tbs, *ans, *tcs)


def kernel(p0, p1, p2, tbox0, tbox1, tbox2, anch0, anch1, anch2,
           tcls0, tcls1, tcls2, b0, b1, b2, a0, a1, a2,
           gj0, gj1, gj2, gi0, gi1, gi2):
    pf = [p0.reshape(-1, 85), p1.reshape(-1, 85), p2.reshape(-1, 85)]
    pad = _NTP - _NT

    gidx, fidx = [], []
    for lvl, (b, a, gj, gi) in enumerate(((b0, a0, gj0, gi0),
                                          (b1, a1, gj1, gi1),
                                          (b2, a2, gj2, gi2))):
        g = _GRIDS[lvl]
        f = ((b.astype(jnp.int32) * _NA + a.astype(jnp.int32)) * g
             + gj.astype(jnp.int32)) * g + gi.astype(jnp.int32)
        f = jnp.concatenate(
            [f, _NCELLS[lvl] + jnp.arange(pad, dtype=jnp.int32)])
        t_ids = jnp.arange(_NTP, dtype=jnp.int32)
        gidx.append(jnp.where(f < _NCELLS[lvl], f,
                              t_ids % _NCELLS[lvl]))
        fidx.append(f.reshape(_IDX_R, 128))
    tval = jnp.arange(_NTP, dtype=jnp.int32).reshape(_IDX_R, 128)

    sc_out = _sc_gather(pf[0], pf[1], pf[2], gidx, fidx, tval)
    gs = sc_out[0:3]
    ws = [w.reshape(_NTP, 1).astype(jnp.float32) for w in sc_out[3:6]]
    obs = [o.reshape(s) for o, s in zip(sc_out[9:12], _OBJ2D)]

    def padf(x, v):
        return jnp.concatenate(
            [x.astype(jnp.float32),
             jnp.full((pad, x.shape[1]), v, jnp.float32)])

    tbs = [padf(t, 1.0) for t in (tbox0, tbox1, tbox2)]
    ans = [padf(t, 1.0) for t in (anch0, anch1, anch2)]
    tcs = [jnp.concatenate([t.astype(jnp.int32),
                            jnp.zeros((pad,), jnp.int32)]).reshape(_NTP, 1)
           for t in (tcls0, tcls1, tcls2)]

    total = _fused(obs, _obj_half(pf[0]), gs, ws, tbs, ans, tcs)
    return total.reshape(())
